# baseline (speedup 1.0000x reference)
import functools

import jax
import jax.numpy as jnp
from jax import lax
from jax.experimental import pallas as pl
from jax.experimental.pallas import tpu as pltpu
from jax.experimental.pallas import tpu_sc as plsc

_T, _M, _E = 50, 10, 4096
_NC, _NS, _L = 2, 16, 16
_NW = _NC * _NS
_RW = (_M * _E) // _NW

_mesh = plsc.VectorSubcoreMesh(core_axis_name="c", subcore_axis_name="s")


@functools.partial(
    pl.kernel,
    out_type=jax.ShapeDtypeStruct((_NW, _RW), jnp.float32),
    mesh=_mesh,
    scratch_types=[
        pltpu.VMEM((_L,), jnp.int32),
        pltpu.VMEM((_L,), jnp.int32),
        pltpu.VMEM((1, _RW), jnp.float32),
        pltpu.SemaphoreType.DMA,
    ],
)
def _pool_lookup(pool_hbm, idv_hbm, out_hbm, idv_v, idx_v, row_v, sem):
    wid = lax.axis_index("s") * _NC + lax.axis_index("c")
    pltpu.sync_copy(idv_hbm, idv_v)
    idx_v[...] = idv_v[...] * _NW + wid
    pltpu.async_copy(pool_hbm.at[idx_v.at[pl.ds(0, 1)]], row_v, sem).wait()
    pltpu.sync_copy(row_v, out_hbm.at[pl.ds(wid, 1)])


def kernel(pool, id):
    pool2 = pool.reshape(_T * _NW, _RW)
    idv = jnp.full((_L,), id, dtype=jnp.int32)
    out = _pool_lookup(pool2, idv)
    return out.reshape(_M, _E)


# speedup vs baseline: 1.6374x; 1.6374x over previous
import functools

import jax
import jax.numpy as jnp
from jax import lax
from jax.experimental import pallas as pl
from jax.experimental.pallas import tpu as pltpu
from jax.experimental.pallas import tpu_sc as plsc

_T, _M, _E = 50, 10, 4096
_NC = 2
_CW = _E // _NC

_mesh = plsc.ScalarSubcoreMesh(axis_name="c", num_cores=_NC)


@functools.partial(
    pl.kernel,
    out_type=jax.ShapeDtypeStruct((_M, _E), jnp.float32),
    mesh=_mesh,
    scratch_types=[
        pltpu.SMEM((1,), jnp.int32),
    ],
)
def _pool_lookup(pool_hbm, idv_hbm, out_hbm, id_s):
    c = lax.axis_index("c")
    pltpu.sync_copy(idv_hbm, id_s)
    i = id_s[0]
    col = c * _CW
    pltpu.sync_copy(
        pool_hbm.at[i, :, pl.ds(col, _CW)],
        out_hbm.at[:, pl.ds(col, _CW)],
    )


def kernel(pool, id):
    idv = jnp.full((1,), id, dtype=jnp.int32)
    return _pool_lookup(pool, idv)


# speedup vs baseline: 1.7057x; 1.0417x over previous
import functools

import jax
import jax.numpy as jnp
from jax import lax
from jax.experimental import pallas as pl
from jax.experimental.pallas import tpu as pltpu
from jax.experimental.pallas import tpu_sc as plsc

_T, _M, _E = 50, 10, 4096
_NC = 1
_CW = _E // _NC

_mesh = plsc.ScalarSubcoreMesh(axis_name="c", num_cores=_NC)


@functools.partial(
    pl.kernel,
    out_type=jax.ShapeDtypeStruct((_M, _E), jnp.float32),
    mesh=_mesh,
    scratch_types=[
        pltpu.SMEM((1,), jnp.int32),
    ],
)
def _pool_lookup(pool_hbm, idv_hbm, out_hbm, id_s):
    c = lax.axis_index("c")
    pltpu.sync_copy(idv_hbm, id_s)
    i = id_s[0]
    col = c * _CW
    pltpu.sync_copy(
        pool_hbm.at[i, :, pl.ds(col, _CW)],
        out_hbm.at[:, pl.ds(col, _CW)],
    )


def kernel(pool, id):
    idv = jnp.full((1,), id, dtype=jnp.int32)
    return _pool_lookup(pool, idv)
